# baseline (speedup 1.0000x reference)
import jax
import jax.numpy as jnp
from jax.experimental import pallas as pl

_SEQ_BLK = 512


def _add_body(w_ref, pe_ref, out_ref):
    out_ref[...] = w_ref[...] + pe_ref[...]


def kernel(weights, pe):
    b, s, d = weights.shape
    n_seq = s // _SEQ_BLK
    return pl.pallas_call(
        _add_body,
        grid=(n_seq, b),
        in_specs=[
            pl.BlockSpec((1, _SEQ_BLK, d), lambda i, j: (j, i, 0)),
            pl.BlockSpec((_SEQ_BLK, d), lambda i, j: (i, 0)),
        ],
        out_specs=pl.BlockSpec((1, _SEQ_BLK, d), lambda i, j: (j, i, 0)),
        out_shape=jax.ShapeDtypeStruct((b, s, d), weights.dtype),
    )(weights, pe)


# speedup vs baseline: 1.1156x; 1.1156x over previous
import jax
import jax.numpy as jnp
from jax.experimental import pallas as pl

_SEQ_BLK = 1024


def _add_body(w_ref, pe_ref, out_ref):
    out_ref[...] = w_ref[...] + pe_ref[...]


def kernel(weights, pe):
    b, s, d = weights.shape
    n_seq = s // _SEQ_BLK
    return pl.pallas_call(
        _add_body,
        grid=(n_seq, b),
        in_specs=[
            pl.BlockSpec((1, _SEQ_BLK, d), lambda i, j: (j, i, 0)),
            pl.BlockSpec((_SEQ_BLK, d), lambda i, j: (i, 0)),
        ],
        out_specs=pl.BlockSpec((1, _SEQ_BLK, d), lambda i, j: (j, i, 0)),
        out_shape=jax.ShapeDtypeStruct((b, s, d), weights.dtype),
    )(weights, pe)


# speedup vs baseline: 1.1606x; 1.0404x over previous
import jax
import jax.numpy as jnp
from jax.experimental import pallas as pl

_SEQ_BLK = 2048


def _add_body(w_ref, pe_ref, out_ref):
    out_ref[...] = w_ref[...] + pe_ref[...]


def kernel(weights, pe):
    b, s, d = weights.shape
    n_seq = s // _SEQ_BLK
    return pl.pallas_call(
        _add_body,
        grid=(n_seq, b),
        in_specs=[
            pl.BlockSpec((1, _SEQ_BLK, d), lambda i, j: (j, i, 0)),
            pl.BlockSpec((_SEQ_BLK, d), lambda i, j: (i, 0)),
        ],
        out_specs=pl.BlockSpec((1, _SEQ_BLK, d), lambda i, j: (j, i, 0)),
        out_shape=jax.ShapeDtypeStruct((b, s, d), weights.dtype),
    )(weights, pe)
